# baseline (speedup 1.0000x reference)
import functools

import jax
import jax.numpy as jnp
from jax import lax
from jax.experimental import pallas as pl
from jax.experimental.pallas import tpu as pltpu
from jax.experimental.pallas import tpu_sc as plsc

_NC = 2
_NS = 16
_NW = _NC * _NS
_CBS = 128
_CBW = 64
_UF = 8


def kernel(state, action, weights):
    B, F1, F2 = state.shape
    F = F1 * F2
    V = weights.shape[0]
    assert B % (_NW * _CBS) == 0 and F % 128 == 0 and F % _UF == 0
    ns_chunks = B // (_NW * _CBS)
    nw_per_s = _CBS // _CBW
    nw_chunks = ns_chunks * nw_per_s
    b_per_w = ns_chunks * _CBS

    stateT = state.transpose(1, 2, 0).reshape(F, B)
    action32 = action.astype(jnp.int32)
    table = weights.reshape(V, F).reshape(V, F // 128, 128)

    mesh = plsc.VectorSubcoreMesh(core_axis_name="c", subcore_axis_name="s")

    @functools.partial(
        pl.kernel,
        mesh=mesh,
        compiler_params=pltpu.CompilerParams(needs_layout_passes=False),
        out_type=jax.ShapeDtypeStruct((B,), jnp.float32),
        scratch_types=[
            pltpu.VMEM((b_per_w,), jnp.int32),
            pltpu.VMEM((_CBW, F // 128, 128), jnp.float32),
            pltpu.VMEM((_CBW, F // 128, 128), jnp.float32),
            pltpu.VMEM((F, _CBS), jnp.float32),
            pltpu.VMEM((F, _CBS), jnp.float32),
            pltpu.VMEM_SHARED((F // 2, _NS * _CBS), jnp.float32),
            pltpu.VMEM((b_per_w,), jnp.float32),
            pltpu.SemaphoreType.DMA,
            pltpu.SemaphoreType.DMA,
            pltpu.SemaphoreType.DMA,
            pltpu.SemaphoreType.DMA,
        ],
    )
    def qtable(state_hbm, action_hbm, table_hbm, out_hbm,
               idx_v, w0, w1, s0, s1, g0, obuf, sw0, sw1, ss0, ss1):
        wid = lax.axis_index("s") * _NC + lax.axis_index("c")
        sid = lax.axis_index("s")
        base = wid * b_per_w
        pltpu.sync_copy(action_hbm.at[pl.ds(base, b_per_w)], idx_v)
        wbufs = ((w0, sw0), (w1, sw1))
        sbufs = ((s0, ss0), (s1, ss1))

        pending_w, pending_s = {}, {}

        def start_w(cw):
            wb, sem = wbufs[cw % 2]
            hs = []
            for q in range(2):
                h = pltpu.make_async_copy(
                    table_hbm.at[idx_v.at[pl.ds(cw * _CBW + q * (_CBW // 2),
                                                _CBW // 2)]],
                    wb.at[pl.ds(q * (_CBW // 2), _CBW // 2)], sem)
                h.start()
                hs.append(h)
            pending_w[cw] = hs

        def fetch_s(cs):
            sb, sem = sbufs[cs % 2]
            reg = g0.at[:, pl.ds(sid * _CBS, _CBS)]
            for q in range(2):
                h1 = pltpu.make_async_copy(
                    state_hbm.at[pl.ds(q * (F // 2), F // 2),
                                 pl.ds(base + cs * _CBS, _CBS)], reg, sem)
                h1.start()
                h1.wait()
                h2 = pltpu.make_async_copy(
                    reg, sb.at[pl.ds(q * (F // 2), F // 2)], sem)
                h2.start()
                h2.wait()
            return sb

        lane = lax.broadcasted_iota(jnp.int32, (16,), 0)
        zf = jnp.zeros((16,), jnp.float32)
        zi = jnp.zeros((16,), jnp.int32)

        start_w(0)
        if nw_chunks > 1:
            start_w(1)

        for cs in range(ns_chunks):
            sb = fetch_s(cs)
            for h in range(nw_per_s):
                cw = cs * nw_per_s + h
                for _h in pending_w.pop(cw):
                    _h.wait()
                wb = wbufs[cw % 2][0]
                for g in range(_CBW // 16):
                    rows = lane + (g * 16)
                    col0 = h * _CBW + g * 16

                    def fbody(i, acc, rows=rows, wb=wb, sb=sb, col0=col0):
                        f0 = i * _UF
                        for u in range(_UF):
                            f = f0 + u
                            c1 = zi + lax.shift_right_logical(f, 7)
                            c2 = zi + lax.bitwise_and(f, 127)
                            w = plsc.load_gather(wb, [rows, c1, c2])
                            s = sb[f, pl.ds(col0, 16)]
                            acc = acc + w * s
                        return acc

                    acc = lax.fori_loop(0, F // _UF, fbody, zf)
                    obuf[pl.ds(cw * _CBW + g * 16, 16)] = acc
                if cw + 2 < nw_chunks:
                    start_w(cw + 2)
        pltpu.sync_copy(obuf, out_hbm.at[pl.ds(base, b_per_w)])

    return qtable(stateT, action32, table)


# speedup vs baseline: 1.1096x; 1.1096x over previous
import functools

import jax
import jax.numpy as jnp
from jax import lax
from jax.experimental import pallas as pl
from jax.experimental.pallas import tpu as pltpu
from jax.experimental.pallas import tpu_sc as plsc

_NC = 2
_NS = 16
_NW = _NC * _NS
_CBS = 128
_CBW = 64
_UF = 8


def _make_qtable(B, F, Fh, f_base, add_prev):
    ns_chunks = B // (_NW * _CBS)
    nw_per_s = _CBS // _CBW
    nw_chunks = ns_chunks * nw_per_s
    b_per_w = ns_chunks * _CBS

    mesh = plsc.VectorSubcoreMesh(core_axis_name="c", subcore_axis_name="s")

    def qtable(*refs):
        if add_prev:
            (state_hbm, action_hbm, table_hbm, prev_hbm, out_hbm,
             idx_v, w0, w1, s0, s1, g0, obuf, sw0, sw1, ss) = refs
        else:
            (state_hbm, action_hbm, table_hbm, out_hbm,
             idx_v, w0, w1, s0, s1, g0, obuf, sw0, sw1, ss) = refs
        wid = lax.axis_index("s") * _NC + lax.axis_index("c")
        sid = lax.axis_index("s")
        base = wid * b_per_w
        pltpu.sync_copy(action_hbm.at[pl.ds(base, b_per_w)], idx_v)
        if add_prev:
            pltpu.sync_copy(prev_hbm.at[pl.ds(base, b_per_w)], obuf)
        wbufs = ((w0, sw0), (w1, sw1))
        sbufs = (s0, s1)

        pending_w = {}

        def start_w(cw):
            wb, sem = wbufs[cw % 2]
            h = pltpu.make_async_copy(
                table_hbm.at[idx_v.at[pl.ds(cw * _CBW, _CBW)]], wb, sem)
            h.start()
            pending_w[cw] = h

        def fetch_s(cs):
            sb = sbufs[cs % 2]
            reg = g0.at[:, pl.ds(sid * _CBS, _CBS)]
            for q in range(2):
                h1 = pltpu.make_async_copy(
                    state_hbm.at[pl.ds(f_base + q * (Fh // 2), Fh // 2),
                                 pl.ds(base + cs * _CBS, _CBS)], reg, ss)
                h1.start()
                h1.wait()
                h2 = pltpu.make_async_copy(
                    reg, sb.at[pl.ds(q * (Fh // 2), Fh // 2)], ss)
                h2.start()
                h2.wait()
            return sb

        lane = lax.broadcasted_iota(jnp.int32, (16,), 0)
        zf = jnp.zeros((16,), jnp.float32)
        zi = jnp.zeros((16,), jnp.int32)

        start_w(0)
        if nw_chunks > 1:
            start_w(1)

        for cs in range(ns_chunks):
            sb = fetch_s(cs)
            for h in range(nw_per_s):
                cw = cs * nw_per_s + h
                pending_w.pop(cw).wait()
                wb = wbufs[cw % 2][0]
                for g in range(_CBW // 16):
                    rows = lane + (g * 16)
                    col0 = h * _CBW + g * 16

                    def fbody(i, acc, rows=rows, wb=wb, sb=sb, col0=col0):
                        f0 = i * _UF
                        for u in range(_UF):
                            f = f0 + u
                            col = zi + f
                            w = plsc.load_gather(wb, [rows, col])
                            s = sb[f, pl.ds(col0, 16)]
                            acc = acc + w * s
                        return acc

                    acc = lax.fori_loop(0, Fh // _UF, fbody, zf)
                    o0 = cw * _CBW + g * 16
                    if add_prev:
                        obuf[pl.ds(o0, 16)] = acc + obuf[pl.ds(o0, 16)]
                    else:
                        obuf[pl.ds(o0, 16)] = acc
                if cw + 2 < nw_chunks:
                    start_w(cw + 2)
        pltpu.sync_copy(obuf, out_hbm.at[pl.ds(base, b_per_w)])

    return functools.partial(
        pl.kernel,
        mesh=mesh,
        compiler_params=pltpu.CompilerParams(needs_layout_passes=False),
        out_type=jax.ShapeDtypeStruct((B,), jnp.float32),
        scratch_types=[
            pltpu.VMEM((b_per_w,), jnp.int32),
            pltpu.VMEM((_CBW, Fh), jnp.float32),
            pltpu.VMEM((_CBW, Fh), jnp.float32),
            pltpu.VMEM((Fh, _CBS), jnp.float32),
            pltpu.VMEM((Fh, _CBS), jnp.float32),
            pltpu.VMEM_SHARED((Fh // 2, _NS * _CBS), jnp.float32),
            pltpu.VMEM((b_per_w,), jnp.float32),
            pltpu.SemaphoreType.DMA,
            pltpu.SemaphoreType.DMA,
            pltpu.SemaphoreType.DMA,
        ],
    )(qtable)


def kernel(state, action, weights):
    B, F1, F2 = state.shape
    F = F1 * F2
    V = weights.shape[0]
    Fh = F // 2
    assert B % (_NW * _CBS) == 0 and Fh % _UF == 0 and Fh % 128 == 0

    stateT = state.transpose(1, 2, 0).reshape(F, B)
    action32 = action.astype(jnp.int32)
    table_a = weights[:, : F1 // 2, :].reshape(V, Fh)
    table_b = weights[:, F1 // 2:, :].reshape(V, Fh)

    pass_a = _make_qtable(B, F, Fh, 0, add_prev=False)
    pass_b = _make_qtable(B, F, Fh, Fh, add_prev=True)

    part = pass_a(stateT, action32, table_a)
    return pass_b(stateT, action32, table_b, part)


# speedup vs baseline: 1.1875x; 1.0702x over previous
import functools

import jax
import jax.numpy as jnp
from jax import lax
from jax.experimental import pallas as pl
from jax.experimental.pallas import tpu as pltpu
from jax.experimental.pallas import tpu_sc as plsc

_NC = 2
_NS = 16
_NW = _NC * _NS
_CBS = 128
_CBW = 64
_UF = 8


def _make_qtable(B, F, Fh, f_base, add_prev):
    ns_chunks = B // (_NW * _CBS)
    nw_per_s = _CBS // _CBW
    nw_chunks = ns_chunks * nw_per_s
    b_per_w = ns_chunks * _CBS

    mesh = plsc.VectorSubcoreMesh(core_axis_name="c", subcore_axis_name="s")

    def qtable(*refs):
        if add_prev:
            (state_hbm, action_hbm, table_hbm, prev_hbm, out_hbm,
             idx_v, w0, w1, s0, s1, obuf, sw0, sw1, ss0, ss1) = refs
        else:
            (state_hbm, action_hbm, table_hbm, out_hbm,
             idx_v, w0, w1, s0, s1, obuf, sw0, sw1, ss0, ss1) = refs
        wid = lax.axis_index("s") * _NC + lax.axis_index("c")
        base = wid * b_per_w
        pltpu.sync_copy(action_hbm.at[pl.ds(base, b_per_w)], idx_v)
        if add_prev:
            pltpu.sync_copy(prev_hbm.at[pl.ds(base, b_per_w)], obuf)
        wbufs = ((w0, sw0), (w1, sw1))
        sbufs = ((s0, ss0), (s1, ss1))

        pending_w = {}
        pending_s = {}

        def start_w(cw):
            wb, sem = wbufs[cw % 2]
            h = pltpu.make_async_copy(
                table_hbm.at[idx_v.at[pl.ds(cw * _CBW, _CBW)]], wb, sem)
            h.start()
            pending_w[cw] = h

        def start_s(cs):
            sb, sem = sbufs[cs % 2]
            h = pltpu.make_async_copy(
                state_hbm.at[pl.ds(f_base, Fh),
                             pl.ds(base + cs * _CBS, _CBS)], sb, sem)
            h.start()
            pending_s[cs] = h

        lane = lax.broadcasted_iota(jnp.int32, (16,), 0)
        zf = jnp.zeros((16,), jnp.float32)
        zi = jnp.zeros((16,), jnp.int32)

        start_s(0)
        if ns_chunks > 1:
            start_s(1)
        start_w(0)
        if nw_chunks > 1:
            start_w(1)

        for cs in range(ns_chunks):
            pending_s.pop(cs).wait()
            sb = sbufs[cs % 2][0]
            for h in range(nw_per_s):
                cw = cs * nw_per_s + h
                pending_w.pop(cw).wait()
                wb = wbufs[cw % 2][0]
                for g in range(_CBW // 16):
                    rows = lane + (g * 16)
                    col0 = h * _CBW + g * 16

                    def fbody(i, acc, rows=rows, wb=wb, sb=sb, col0=col0):
                        f0 = i * _UF
                        for u in range(_UF):
                            f = f0 + u
                            col = zi + f
                            w = plsc.load_gather(wb, [rows, col])
                            s = sb[f, pl.ds(col0, 16)]
                            acc = acc + w * s
                        return acc

                    acc = lax.fori_loop(0, Fh // _UF, fbody, zf)
                    o0 = cw * _CBW + g * 16
                    if add_prev:
                        obuf[pl.ds(o0, 16)] = acc + obuf[pl.ds(o0, 16)]
                    else:
                        obuf[pl.ds(o0, 16)] = acc
                if cw + 2 < nw_chunks:
                    start_w(cw + 2)
            if cs + 2 < ns_chunks:
                start_s(cs + 2)
        pltpu.sync_copy(obuf, out_hbm.at[pl.ds(base, b_per_w)])

    return functools.partial(
        pl.kernel,
        mesh=mesh,
        compiler_params=pltpu.CompilerParams(needs_layout_passes=False),
        out_type=jax.ShapeDtypeStruct((B,), jnp.float32),
        scratch_types=[
            pltpu.VMEM((b_per_w,), jnp.int32),
            pltpu.VMEM((_CBW, Fh), jnp.float32),
            pltpu.VMEM((_CBW, Fh), jnp.float32),
            pltpu.VMEM((Fh, _CBS), jnp.float32),
            pltpu.VMEM((Fh, _CBS), jnp.float32),
            pltpu.VMEM((b_per_w,), jnp.float32),
            pltpu.SemaphoreType.DMA,
            pltpu.SemaphoreType.DMA,
            pltpu.SemaphoreType.DMA,
            pltpu.SemaphoreType.DMA,
        ],
    )(qtable)


def kernel(state, action, weights):
    B, F1, F2 = state.shape
    F = F1 * F2
    V = weights.shape[0]
    Fh = F // 2
    assert B % (_NW * _CBS) == 0 and Fh % _UF == 0 and Fh % 128 == 0

    stateT = state.transpose(1, 2, 0).reshape(F, B)
    action32 = action.astype(jnp.int32)
    table_a = weights[:, : F1 // 2, :].reshape(V, Fh)
    table_b = weights[:, F1 // 2:, :].reshape(V, Fh)

    pass_a = _make_qtable(B, F, Fh, 0, add_prev=False)
    pass_b = _make_qtable(B, F, Fh, Fh, add_prev=True)

    part = pass_a(stateT, action32, table_a)
    return pass_b(stateT, action32, table_b, part)
